# baseline (speedup 1.0000x reference)
import jax
import jax.numpy as jnp
from jax import lax
from jax.experimental import pallas as pl
from jax.experimental.pallas import tpu as pltpu
from jax.experimental.pallas import tpu_sc as plsc

B = 16384
BAG = 20
D = 64
V = 100000
L = 16

_INFO = plsc.get_sparse_core_info()
NC, NS = _INFO.num_cores, _INFO.num_subcores
NW = NC * NS
BPW = B // NW

C = 32
NCHUNK = BPW // C
IDX_PER_CHUNK = C * BAG
STREAM = 128
NSTREAM = IDX_PER_CHUNK // STREAM

XF_PAD = 2_228_224
XCH = 64
VCHUNK = 160
NVCH_TOT = V // VCHUNK
NVCH_PW = -(-NVCH_TOT // NW)


def _linearize_table_body(x_hbm, table_hbm, xf_out, tab_out,
                          xstage, xflat, tbufs, flats, sems, wsems):
    wid = lax.axis_index("s") * NC + lax.axis_index("c")
    wbase = wid * NVCH_PW

    def xchunk_body(xc, carry):
        pltpu.sync_copy(x_hbm.at[pl.ds(wid * BPW + xc * XCH, XCH)], xstage)

        def xrow_body(r4, c2):
            for dr in range(4):
                r = r4 * 4 + dr
                tail = xstage[r, pl.ds(BAG - L, L)]
                head = xstage[r, pl.ds(0, L)]
                xflat[pl.ds(r * BAG + (BAG - L), L)] = tail
                xflat[pl.ds(r * BAG, L)] = head
            return c2

        lax.fori_loop(0, XCH // 4, xrow_body, 0)
        pltpu.sync_copy(
            xflat, xf_out.at[pl.ds((wid * BPW + xc * XCH) * BAG, XCH * BAG)])
        return carry

    lax.fori_loop(0, BPW // XCH, xchunk_body, 0)

    def start_read(c, buf):
        return pltpu.async_copy(
            table_hbm.at[pl.ds(c * VCHUNK, VCHUNK)], tbufs.at[buf],
            sems.at[buf])

    def wait_read(buf):
        pltpu.make_async_copy(
            table_hbm.at[pl.ds(0, VCHUNK)], tbufs.at[buf], sems.at[buf]
        ).wait()

    def start_write(c, buf):
        return pltpu.async_copy(
            flats.at[buf],
            tab_out.at[pl.ds(c * (VCHUNK * D), VCHUNK * D)],
            wsems.at[buf])

    def wait_write(buf):
        pltpu.make_async_copy(
            flats.at[buf], tab_out.at[pl.ds(0, VCHUNK * D)],
            wsems.at[buf]).wait()

    def repack(buf):
        tb = tbufs.at[buf]
        fl = flats.at[buf]

        def row_body(r4, carry):
            for dr in range(4):
                r = r4 * 4 + dr
                base = r * D
                for j in range(D // L):
                    fl[pl.ds(base + L * j, L)] = tb[r, pl.ds(L * j, L)]
            return carry

        lax.fori_loop(0, VCHUNK // 4, row_body, 0)

    @pl.when(wbase < NVCH_TOT)
    def _():
        start_read(wbase, 0)

    def chunk_body(g, carry):
        b = lax.rem(g, 2)
        c = wbase + g

        @pl.when(c + 1 < wbase + NVCH_PW)
        def _():
            @pl.when(c + 1 < NVCH_TOT)
            def _():
                start_read(c + 1, 1 - b)

        @pl.when(c < NVCH_TOT)
        def _():
            wait_read(b)

        @pl.when(jnp.logical_and(g >= 2, c - 2 < NVCH_TOT))
        def _():
            wait_write(b)

        repack(b)

        @pl.when(c < NVCH_TOT)
        def _():
            start_write(c, b)

        return carry

    lax.fori_loop(0, NVCH_PW, chunk_body, 0)

    @pl.when(wbase + NVCH_PW - 2 < NVCH_TOT)
    def _():
        wait_write(lax.rem(NVCH_PW - 2, 2))

    @pl.when(wbase + NVCH_PW - 1 < NVCH_TOT)
    def _():
        wait_write(lax.rem(NVCH_PW - 1, 2))


@jax.jit
def _linearize(x, table):
    mesh = plsc.VectorSubcoreMesh(core_axis_name="c", subcore_axis_name="s")
    return pl.kernel(
        _linearize_table_body,
        out_type=(
            pltpu.HBM((XF_PAD,), jnp.int32),
            pltpu.HBM((V * D,), jnp.float32),
        ),
        mesh=mesh,
        scratch_types=[
            pltpu.VMEM((XCH, BAG), jnp.int32),
            pltpu.VMEM((XCH * BAG,), jnp.int32),
            pltpu.VMEM((2, VCHUNK, D), jnp.float32),
            pltpu.VMEM((2, VCHUNK * D), jnp.float32),
            pltpu.SemaphoreType.DMA((2,)),
            pltpu.SemaphoreType.DMA((2,)),
        ],
        compiler_params=pltpu.CompilerParams(use_tc_tiling_on_sc=True),
    )(x, table)


def _bag_sum_body(xf_hbm, tab_hbm, out_hbm, idx_v, rows_v, outb, sems):
    wid = lax.axis_index("s") * NC + lax.axis_index("c")
    pltpu.sync_copy(xf_hbm.at[pl.ds(wid * BPW * BAG, BPW * BAG)], idx_v)

    def start_gathers(g, buf):
        return [
            pltpu.async_copy(
                tab_hbm.at[idx_v.at[pl.ds(g * IDX_PER_CHUNK + j * STREAM, STREAM)]],
                rows_v.at[buf].at[pl.ds(j * STREAM, STREAM)],
                sems.at[buf],
            )
            for j in range(NSTREAM)
        ]

    def wait_gathers(buf):
        for j in range(NSTREAM):
            pltpu.make_async_copy(
                tab_hbm.at[idx_v.at[pl.ds(j * STREAM, STREAM)]],
                rows_v.at[buf].at[pl.ds(j * STREAM, STREAM)],
                sems.at[buf],
            ).wait()

    def compute_chunk(g, buf):
        rv = rows_v.at[buf]

        def bag_body(b2, c2):
            r0 = b2 * (2 * BAG)
            accs = [
                rv[r0 + b_off * BAG, pl.ds(L * j, L)]
                for b_off in range(2)
                for j in range(D // L)
            ]
            for l in range(1, BAG):
                for k, (b_off, j) in enumerate(
                    (b, j) for b in range(2) for j in range(D // L)
                ):
                    accs[k] = accs[k] + rv[r0 + b_off * BAG + l, pl.ds(L * j, L)]
            for k, (b_off, j) in enumerate(
                (b, j) for b in range(2) for j in range(D // L)
            ):
                outb[2 * b2 + b_off, pl.ds(L * j, L)] = accs[k]
            return c2

        lax.fori_loop(0, C // 2, bag_body, 0)
        pltpu.sync_copy(outb, out_hbm.at[pl.ds(wid * BPW + g * C, C)])

    start_gathers(0, 0)

    def pair_body(g2, carry):
        g0 = 2 * g2
        start_gathers(g0 + 1, 1)
        wait_gathers(0)
        compute_chunk(g0, 0)
        start_gathers(g0 + 2, 0)
        wait_gathers(1)
        compute_chunk(g0 + 1, 1)
        return carry

    lax.fori_loop(0, NCHUNK // 2 - 1, pair_body, 0)
    start_gathers(NCHUNK - 1, 1)
    wait_gathers(0)
    compute_chunk(NCHUNK - 2, 0)
    wait_gathers(1)
    compute_chunk(NCHUNK - 1, 1)


@jax.jit
def _bag_sum(xf, tab2d):
    mesh = plsc.VectorSubcoreMesh(core_axis_name="c", subcore_axis_name="s")
    return pl.kernel(
        _bag_sum_body,
        out_type=jax.ShapeDtypeStruct((B, D), jnp.float32),
        mesh=mesh,
        scratch_types=[
            pltpu.VMEM((BPW * BAG,), jnp.int32),
            pltpu.VMEM((2, IDX_PER_CHUNK, D), jnp.float32),
            pltpu.VMEM((C, D), jnp.float32),
            pltpu.SemaphoreType.DMA((2,)),
        ],
        compiler_params=pltpu.CompilerParams(use_tc_tiling_on_sc=False),
    )(xf, tab2d)


def kernel(X, table):
    xf, tab1d = _linearize(X.astype(jnp.int32), table)
    tab2d = tab1d.reshape(V, D)
    return _bag_sum(xf, tab2d)


# speedup vs baseline: 1.2747x; 1.2747x over previous
import functools

import jax
import jax.numpy as jnp
from jax import lax
from jax.experimental import pallas as pl
from jax.experimental.pallas import tpu as pltpu
from jax.experimental.pallas import tpu_sc as plsc

B = 16384
BAG = 20
D = 64
L = 16

_INFO = plsc.get_sparse_core_info()
NC, NS = _INFO.num_cores, _INFO.num_subcores
NW = NC * NS
BPW = B // NW

C = 32
NCHUNK = BPW // C
IDX_PER_CHUNK = C * BAG
STREAM = 128
NSTREAM = IDX_PER_CHUNK // STREAM
IDX_ROWS_PER_W = (B * BAG) // (NW * STREAM)


def _bag_sum_body(x_hbm, table_hbm, out_hbm, xstage, idx_v, rows_v, outb, sems):
    wid = lax.axis_index("s") * NC + lax.axis_index("c")

    pltpu.sync_copy(x_hbm.at[pl.ds(wid * BPW, BPW)], xstage)

    def xrow_body(r4, c2):
        for dr in range(4):
            r = r4 * 4 + dr
            tail = xstage[r, pl.ds(BAG - L, L)]
            head = xstage[r, pl.ds(0, L)]
            idx_v[pl.ds(r * BAG + (BAG - L), L)] = tail
            idx_v[pl.ds(r * BAG, L)] = head
        return c2

    lax.fori_loop(0, BPW // 4, xrow_body, 0)

    def start_gathers(g, buf):
        return [
            pltpu.async_copy(
                table_hbm.at[idx_v.at[pl.ds(g * IDX_PER_CHUNK + j * STREAM, STREAM)]],
                rows_v.at[buf].at[pl.ds(j * STREAM, STREAM)],
                sems.at[buf],
            )
            for j in range(NSTREAM)
        ]

    def wait_gathers(buf):
        for j in range(NSTREAM):
            pltpu.make_async_copy(
                table_hbm.at[idx_v.at[pl.ds(j * STREAM, STREAM)]],
                rows_v.at[buf].at[pl.ds(j * STREAM, STREAM)],
                sems.at[buf],
            ).wait()

    def compute_chunk(g, buf):
        rv = rows_v.at[buf]

        def bag_body(b2, c2):
            r0 = b2 * (2 * BAG)
            accs = [
                rv[r0 + b_off * BAG, pl.ds(L * j, L)]
                for b_off in range(2)
                for j in range(D // L)
            ]
            for l in range(1, BAG):
                for k, (b_off, j) in enumerate(
                    (b, j) for b in range(2) for j in range(D // L)
                ):
                    accs[k] = accs[k] + rv[r0 + b_off * BAG + l, pl.ds(L * j, L)]
            for k, (b_off, j) in enumerate(
                (b, j) for b in range(2) for j in range(D // L)
            ):
                outb[2 * b2 + b_off, pl.ds(L * j, L)] = accs[k]
            return c2

        lax.fori_loop(0, C // 2, bag_body, 0)
        pltpu.sync_copy(outb, out_hbm.at[pl.ds(wid * BPW + g * C, C)])

    start_gathers(0, 0)

    def pair_body(g2, carry):
        g0 = 2 * g2
        start_gathers(g0 + 1, 1)
        wait_gathers(0)
        compute_chunk(g0, 0)
        start_gathers(g0 + 2, 0)
        wait_gathers(1)
        compute_chunk(g0 + 1, 1)
        return carry

    lax.fori_loop(0, NCHUNK // 2 - 1, pair_body, 0)
    start_gathers(NCHUNK - 1, 1)
    wait_gathers(0)
    compute_chunk(NCHUNK - 2, 0)
    wait_gathers(1)
    compute_chunk(NCHUNK - 1, 1)


@jax.jit
def _bag_sum(x, table):
    mesh = plsc.VectorSubcoreMesh(core_axis_name="c", subcore_axis_name="s")
    return pl.kernel(
        _bag_sum_body,
        out_type=jax.ShapeDtypeStruct((B, D), jnp.float32),
        mesh=mesh,
        scratch_types=[
            pltpu.VMEM((BPW, BAG), jnp.int32),
            pltpu.VMEM((BPW * BAG,), jnp.int32),
            pltpu.VMEM((2, IDX_PER_CHUNK, D), jnp.float32),
            pltpu.VMEM((C, D), jnp.float32),
            pltpu.SemaphoreType.DMA((2,)),
        ],
        compiler_params=pltpu.CompilerParams(use_tc_tiling_on_sc=False),
    )(x, table)


def kernel(X, table):
    return _bag_sum(X.astype(jnp.int32), table)


# speedup vs baseline: 1.3665x; 1.0720x over previous
import functools

import jax
import jax.numpy as jnp
from jax import lax
from jax.experimental import pallas as pl
from jax.experimental.pallas import tpu as pltpu
from jax.experimental.pallas import tpu_sc as plsc

B = 16384
BAG = 20
D = 64
L = 16

_INFO = plsc.get_sparse_core_info()
NC, NS = _INFO.num_cores, _INFO.num_subcores
NW = NC * NS
BPW = B // NW

C = 32
NCHUNK = BPW // C
IDX_PER_CHUNK = C * BAG
STREAM = 128
NSTREAM = IDX_PER_CHUNK // STREAM
IDX_ROWS_PER_W = (B * BAG) // (NW * STREAM)


def _bag_sum_body(xf_hbm, table_hbm, out_hbm, idx_v, rows_v, outb, sems):
    wid = lax.axis_index("s") * NC + lax.axis_index("c")
    pltpu.sync_copy(
        xf_hbm.at[pl.ds(wid * (BPW * BAG), BPW * BAG)], idx_v)

    def start_gathers(g, buf):
        return [
            pltpu.async_copy(
                table_hbm.at[idx_v.at[pl.ds(g * IDX_PER_CHUNK + j * STREAM, STREAM)]],
                rows_v.at[buf].at[pl.ds(j * STREAM, STREAM)],
                sems.at[buf],
            )
            for j in range(NSTREAM)
        ]

    def wait_gathers(buf):
        for j in range(NSTREAM):
            pltpu.make_async_copy(
                table_hbm.at[idx_v.at[pl.ds(j * STREAM, STREAM)]],
                rows_v.at[buf].at[pl.ds(j * STREAM, STREAM)],
                sems.at[buf],
            ).wait()

    def compute_chunk(g, buf):
        rv = rows_v.at[buf]

        def bag_body(b2, c2):
            r0 = b2 * (2 * BAG)
            accs = [
                rv[r0 + b_off * BAG, pl.ds(L * j, L)]
                for b_off in range(2)
                for j in range(D // L)
            ]
            for l in range(1, BAG):
                for k, (b_off, j) in enumerate(
                    (b, j) for b in range(2) for j in range(D // L)
                ):
                    accs[k] = accs[k] + rv[r0 + b_off * BAG + l, pl.ds(L * j, L)]
            for k, (b_off, j) in enumerate(
                (b, j) for b in range(2) for j in range(D // L)
            ):
                outb[2 * b2 + b_off, pl.ds(L * j, L)] = accs[k]
            return c2

        lax.fori_loop(0, C // 2, bag_body, 0)
        pltpu.sync_copy(outb, out_hbm.at[pl.ds(wid * BPW + g * C, C)])

    start_gathers(0, 0)

    def pair_body(g2, carry):
        g0 = 2 * g2
        start_gathers(g0 + 1, 1)
        wait_gathers(0)
        compute_chunk(g0, 0)
        start_gathers(g0 + 2, 0)
        wait_gathers(1)
        compute_chunk(g0 + 1, 1)
        return carry

    lax.fori_loop(0, NCHUNK // 2 - 1, pair_body, 0)
    start_gathers(NCHUNK - 1, 1)
    wait_gathers(0)
    compute_chunk(NCHUNK - 2, 0)
    wait_gathers(1)
    compute_chunk(NCHUNK - 1, 1)


@jax.jit
def _bag_sum(x, table):
    mesh = plsc.VectorSubcoreMesh(core_axis_name="c", subcore_axis_name="s")
    return pl.kernel(
        _bag_sum_body,
        out_type=jax.ShapeDtypeStruct((B, D), jnp.float32),
        mesh=mesh,
        scratch_types=[
            pltpu.VMEM((BPW * BAG,), jnp.int32),
            pltpu.VMEM((2, IDX_PER_CHUNK, D), jnp.float32),
            pltpu.VMEM((C, D), jnp.float32),
            pltpu.SemaphoreType.DMA((2,)),
        ],
        compiler_params=pltpu.CompilerParams(use_tc_tiling_on_sc=False),
    )(x, table)


def kernel(X, table):
    return _bag_sum(X.astype(jnp.int32).reshape(B * BAG), table)


# speedup vs baseline: 1.3736x; 1.0052x over previous
import functools

import jax
import jax.numpy as jnp
from jax import lax
from jax.experimental import pallas as pl
from jax.experimental.pallas import tpu as pltpu
from jax.experimental.pallas import tpu_sc as plsc

B = 16384
BAG = 20
D = 64
L = 16

_INFO = plsc.get_sparse_core_info()
NC, NS = _INFO.num_cores, _INFO.num_subcores
NW = NC * NS
BPW = B // NW

C = 32
NCHUNK = BPW // C
IDX_PER_CHUNK = C * BAG
STREAM = 640
NSTREAM = IDX_PER_CHUNK // STREAM
IDX_ROWS_PER_W = (B * BAG) // (NW * STREAM)


def _bag_sum_body(xf_hbm, table_hbm, out_hbm, idx_v, rows_v, outb, sems):
    wid = lax.axis_index("s") * NC + lax.axis_index("c")
    pltpu.sync_copy(
        xf_hbm.at[pl.ds(wid * (BPW * BAG), BPW * BAG)], idx_v)

    def start_gathers(g, buf):
        return [
            pltpu.async_copy(
                table_hbm.at[idx_v.at[pl.ds(g * IDX_PER_CHUNK + j * STREAM, STREAM)]],
                rows_v.at[buf].at[pl.ds(j * STREAM, STREAM)],
                sems.at[buf],
            )
            for j in range(NSTREAM)
        ]

    def wait_gathers(buf):
        for j in range(NSTREAM):
            pltpu.make_async_copy(
                table_hbm.at[idx_v.at[pl.ds(j * STREAM, STREAM)]],
                rows_v.at[buf].at[pl.ds(j * STREAM, STREAM)],
                sems.at[buf],
            ).wait()

    def compute_chunk(g, buf):
        rv = rows_v.at[buf]

        def bag_body(b2, c2):
            r0 = b2 * (2 * BAG)
            accs = [
                rv[r0 + b_off * BAG, pl.ds(L * j, L)]
                for b_off in range(2)
                for j in range(D // L)
            ]
            for l in range(1, BAG):
                for k, (b_off, j) in enumerate(
                    (b, j) for b in range(2) for j in range(D // L)
                ):
                    accs[k] = accs[k] + rv[r0 + b_off * BAG + l, pl.ds(L * j, L)]
            for k, (b_off, j) in enumerate(
                (b, j) for b in range(2) for j in range(D // L)
            ):
                outb[2 * b2 + b_off, pl.ds(L * j, L)] = accs[k]
            return c2

        lax.fori_loop(0, C // 2, bag_body, 0)
        pltpu.sync_copy(outb, out_hbm.at[pl.ds(wid * BPW + g * C, C)])

    start_gathers(0, 0)

    def pair_body(g2, carry):
        g0 = 2 * g2
        start_gathers(g0 + 1, 1)
        wait_gathers(0)
        compute_chunk(g0, 0)
        start_gathers(g0 + 2, 0)
        wait_gathers(1)
        compute_chunk(g0 + 1, 1)
        return carry

    lax.fori_loop(0, NCHUNK // 2 - 1, pair_body, 0)
    start_gathers(NCHUNK - 1, 1)
    wait_gathers(0)
    compute_chunk(NCHUNK - 2, 0)
    wait_gathers(1)
    compute_chunk(NCHUNK - 1, 1)


@jax.jit
def _bag_sum(x, table):
    mesh = plsc.VectorSubcoreMesh(core_axis_name="c", subcore_axis_name="s")
    return pl.kernel(
        _bag_sum_body,
        out_type=jax.ShapeDtypeStruct((B, D), jnp.float32),
        mesh=mesh,
        scratch_types=[
            pltpu.VMEM((BPW * BAG,), jnp.int32),
            pltpu.VMEM((2, IDX_PER_CHUNK, D), jnp.float32),
            pltpu.VMEM((C, D), jnp.float32),
            pltpu.SemaphoreType.DMA((2,)),
        ],
        compiler_params=pltpu.CompilerParams(use_tc_tiling_on_sc=False),
    )(x, table)


def kernel(X, table):
    return _bag_sum(X.astype(jnp.int32).reshape(B * BAG), table)
